# Initial kernel scaffold; baseline (speedup 1.0000x reference)
import jax
import jax.numpy as jnp
from jax.experimental import pallas as pl


def kernel(wav_input, graph_features, graph_input, gcn1_W, gcn1_b, gcn2_W, gcn2_b, fc1_W, fc1_b, pga_W, pga_b, pgv_W, pgv_b, sa03_W, sa03_b, sa10_W, sa10_b, sa30_W, sa30_b):
    raise NotImplementedError("write your pallas kernel here")


# speedup vs baseline: 19.8750x; 19.8750x over previous
import functools

import jax
import jax.numpy as jnp
from jax.experimental import pallas as pl

N_NODES = 39
F_IN = 12
HID = 64
FC = 128
N_OUT = 5 * N_NODES


def _norm_adj_t(a, at):
    deg_row = jnp.sum(a, axis=0, keepdims=True)
    deg_col = jnp.sum(at, axis=1, keepdims=True)
    dr = 1.0 / jnp.sqrt(jnp.maximum(deg_row, 1e-12))
    dc = 1.0 / jnp.sqrt(jnp.maximum(deg_col, 1e-12))
    return dc * at * dr


def _mix_kernel(a_ref, at_ref, x_ref, o_ref):
    mt = _norm_adj_t(a_ref[...], at_ref[...])
    o_ref[...] = jnp.dot(mt, x_ref[...], preferred_element_type=jnp.float32)


def _lin_kernel(x_ref, w1_ref, b1_ref, w2_ref, o_ref):
    h1 = jnp.dot(x_ref[...], w1_ref[...], preferred_element_type=jnp.float32)
    x1 = jax.nn.relu(h1 + b1_ref[...])
    o_ref[...] = jnp.dot(x1, w2_ref[...], preferred_element_type=jnp.float32)


def _head_kernel(x_ref, b2_ref, w3_ref, b3_ref, wh_ref, bh_ref, o_ref, *, bs):
    y = jnp.zeros((bs, FC), dtype=jnp.float32)
    for n in range(N_NODES):
        x2n = jnp.tanh(x_ref[n] + b2_ref[...])
        y = y + jnp.dot(x2n, w3_ref[n], preferred_element_type=jnp.float32)
    y = jax.nn.relu(y + b3_ref[...])
    o_ref[...] = jnp.dot(y, wh_ref[...], preferred_element_type=jnp.float32) + bh_ref[...]


def _full(shape):
    return pl.BlockSpec(shape, lambda *_: tuple(0 for _ in shape))


def kernel(wav_input, graph_features, graph_input, gcn1_W, gcn1_b, gcn2_W,
           gcn2_b, fc1_W, fc1_b, pga_W, pga_b, pgv_W, pgv_b, sa03_W, sa03_b,
           sa10_W, sa10_b, sa30_W, sa30_b):
    del graph_features
    B = wav_input.shape[0]
    f32 = jnp.float32

    xt = jnp.transpose(wav_input, (1, 0, 2)).reshape(N_NODES, B * F_IN)
    a = graph_input[0]
    at = a.T
    b1 = gcn1_b.reshape(1, HID)
    b2 = gcn2_b.reshape(1, HID)
    w3 = fc1_W.reshape(N_NODES, HID, FC)
    b3 = fc1_b.reshape(1, FC)
    wh = jnp.concatenate([pga_W, pgv_W, sa03_W, sa10_W, sa30_W], axis=1)
    bh = jnp.concatenate([pga_b, pgv_b, sa03_b, sa10_b, sa30_b]).reshape(1, N_OUT)

    xm = pl.pallas_call(
        _mix_kernel,
        in_specs=[_full((N_NODES, N_NODES)), _full((N_NODES, N_NODES)),
                  _full((N_NODES, B * F_IN))],
        out_specs=_full((N_NODES, B * F_IN)),
        out_shape=jax.ShapeDtypeStruct((N_NODES, B * F_IN), f32),
    )(a, at, xt)

    rows = N_NODES * B
    rbs = rows // 8
    h2 = pl.pallas_call(
        _lin_kernel,
        grid=(8,),
        in_specs=[pl.BlockSpec((rbs, F_IN), lambda i: (i, 0)),
                  _full((F_IN, HID)), _full((1, HID)), _full((HID, HID))],
        out_specs=pl.BlockSpec((rbs, HID), lambda i: (i, 0)),
        out_shape=jax.ShapeDtypeStruct((rows, HID), f32),
    )(xm.reshape(rows, F_IN), gcn1_W, b1, gcn2_W)

    lbs = B * HID // 4
    x2m = pl.pallas_call(
        _mix_kernel,
        grid=(4,),
        in_specs=[_full((N_NODES, N_NODES)), _full((N_NODES, N_NODES)),
                  pl.BlockSpec((N_NODES, lbs), lambda i: (0, i))],
        out_specs=pl.BlockSpec((N_NODES, lbs), lambda i: (0, i)),
        out_shape=jax.ShapeDtypeStruct((N_NODES, B * HID), f32),
    )(a, at, h2.reshape(N_NODES, B * HID))

    bs = 256
    out = pl.pallas_call(
        functools.partial(_head_kernel, bs=bs),
        grid=(B // bs,),
        in_specs=[pl.BlockSpec((N_NODES, bs, HID), lambda i: (0, i, 0)),
                  _full((1, HID)), _full((N_NODES, HID, FC)), _full((1, FC)),
                  _full((FC, N_OUT)), _full((1, N_OUT))],
        out_specs=pl.BlockSpec((bs, N_OUT), lambda i: (i, 0)),
        out_shape=jax.ShapeDtypeStruct((B, N_OUT), f32),
    )(x2m.reshape(N_NODES, B, HID), b2, w3, b3, wh, bh)

    return (out[:, 0:39], out[:, 39:78], out[:, 78:117],
            out[:, 117:156], out[:, 156:195])


# speedup vs baseline: 23.2876x; 1.1717x over previous
import functools

import jax
import jax.numpy as jnp
from jax.experimental import pallas as pl

N_NODES = 39
F_IN = 12
HID = 64
FC = 128
N_OUT = 5 * N_NODES


def _norm_adj_t(a, at):
    deg_row = jnp.sum(a, axis=0, keepdims=True)
    deg_col = jnp.sum(at, axis=1, keepdims=True)
    dr = 1.0 / jnp.sqrt(jnp.maximum(deg_row, 1e-12))
    dc = 1.0 / jnp.sqrt(jnp.maximum(deg_col, 1e-12))
    return dc * at * dr


def _mix_kernel(a_ref, at_ref, x_ref, o_ref):
    mt = _norm_adj_t(a_ref[...], at_ref[...]).astype(jnp.bfloat16)
    o_ref[...] = jnp.dot(mt, x_ref[...],
                         preferred_element_type=jnp.float32).astype(jnp.bfloat16)


def _lin_kernel(x_ref, w1_ref, b1_ref, w2_ref, o_ref):
    w1 = w1_ref[...].astype(jnp.bfloat16)
    w2 = w2_ref[...].astype(jnp.bfloat16)
    h1 = jnp.dot(x_ref[...], w1, preferred_element_type=jnp.float32)
    x1 = jax.nn.relu(h1 + b1_ref[...]).astype(jnp.bfloat16)
    o_ref[...] = jnp.dot(x1, w2,
                         preferred_element_type=jnp.float32).astype(jnp.bfloat16)


def _head_kernel(x_ref, b2_ref, w3_ref, b3_ref, wh_ref, bh_ref,
                 o0_ref, o1_ref, o2_ref, o3_ref, o4_ref, *, bs):
    y = jnp.zeros((bs, FC), dtype=jnp.float32)
    for n in range(N_NODES):
        x2n = jnp.tanh(x_ref[n].astype(jnp.float32) + b2_ref[...])
        y = y + jnp.dot(x2n.astype(jnp.bfloat16),
                        w3_ref[n].astype(jnp.bfloat16),
                        preferred_element_type=jnp.float32)
    y = jax.nn.relu(y + b3_ref[...]).astype(jnp.bfloat16)
    out = jnp.dot(y, wh_ref[...].astype(jnp.bfloat16),
                  preferred_element_type=jnp.float32) + bh_ref[...]
    o0_ref[...] = out[:, 0 * N_NODES:1 * N_NODES]
    o1_ref[...] = out[:, 1 * N_NODES:2 * N_NODES]
    o2_ref[...] = out[:, 2 * N_NODES:3 * N_NODES]
    o3_ref[...] = out[:, 3 * N_NODES:4 * N_NODES]
    o4_ref[...] = out[:, 4 * N_NODES:5 * N_NODES]


def _full(shape):
    return pl.BlockSpec(shape, lambda *_: tuple(0 for _ in shape))


def kernel(wav_input, graph_features, graph_input, gcn1_W, gcn1_b, gcn2_W,
           gcn2_b, fc1_W, fc1_b, pga_W, pga_b, pgv_W, pgv_b, sa03_W, sa03_b,
           sa10_W, sa10_b, sa30_W, sa30_b):
    del graph_features
    B = wav_input.shape[0]
    f32 = jnp.float32
    bf16 = jnp.bfloat16

    xt = jnp.transpose(wav_input, (1, 0, 2)).reshape(N_NODES, B * F_IN)
    xt = xt.astype(bf16)
    a = graph_input[0]
    at = a.T
    b1 = gcn1_b.reshape(1, HID)
    b2 = gcn2_b.reshape(1, HID)
    w3 = fc1_W.reshape(N_NODES, HID, FC)
    b3 = fc1_b.reshape(1, FC)
    wh = jnp.concatenate([pga_W, pgv_W, sa03_W, sa10_W, sa30_W], axis=1)
    bh = jnp.concatenate([pga_b, pgv_b, sa03_b, sa10_b, sa30_b]).reshape(1, N_OUT)

    xm = pl.pallas_call(
        _mix_kernel,
        in_specs=[_full((N_NODES, N_NODES)), _full((N_NODES, N_NODES)),
                  _full((N_NODES, B * F_IN))],
        out_specs=_full((N_NODES, B * F_IN)),
        out_shape=jax.ShapeDtypeStruct((N_NODES, B * F_IN), bf16),
    )(a, at, xt)

    rows = N_NODES * B
    rbs = rows // 8
    h2 = pl.pallas_call(
        _lin_kernel,
        grid=(8,),
        in_specs=[pl.BlockSpec((rbs, F_IN), lambda i: (i, 0)),
                  _full((F_IN, HID)), _full((1, HID)), _full((HID, HID))],
        out_specs=pl.BlockSpec((rbs, HID), lambda i: (i, 0)),
        out_shape=jax.ShapeDtypeStruct((rows, HID), bf16),
    )(xm.reshape(rows, F_IN), gcn1_W, b1, gcn2_W)

    lbs = B * HID // 4
    x2m = pl.pallas_call(
        _mix_kernel,
        grid=(4,),
        in_specs=[_full((N_NODES, N_NODES)), _full((N_NODES, N_NODES)),
                  pl.BlockSpec((N_NODES, lbs), lambda i: (0, i))],
        out_specs=pl.BlockSpec((N_NODES, lbs), lambda i: (0, i)),
        out_shape=jax.ShapeDtypeStruct((N_NODES, B * HID), bf16),
    )(a, at, h2.reshape(N_NODES, B * HID))

    bs = 256
    out_sds = jax.ShapeDtypeStruct((B, N_NODES), f32)
    out_spec = pl.BlockSpec((bs, N_NODES), lambda i: (i, 0))
    outs = pl.pallas_call(
        functools.partial(_head_kernel, bs=bs),
        grid=(B // bs,),
        in_specs=[pl.BlockSpec((N_NODES, bs, HID), lambda i: (0, i, 0)),
                  _full((1, HID)), _full((N_NODES, HID, FC)), _full((1, FC)),
                  _full((FC, N_OUT)), _full((1, N_OUT))],
        out_specs=[out_spec] * 5,
        out_shape=[out_sds] * 5,
    )(x2m.reshape(N_NODES, B, HID), b2, w3, b3, wh, bh)

    return tuple(outs)


# speedup vs baseline: 40.8973x; 1.7562x over previous
import functools

import jax
import jax.numpy as jnp
from jax import lax
from jax.experimental import pallas as pl

N_NODES = 39
F_IN = 12
HID = 64
FC = 128

def _norm_adj_t(a, at):
    deg_row = jnp.sum(a, axis=0, keepdims=True)
    deg_col = jnp.sum(at, axis=1, keepdims=True)
    dr = 1.0 / jnp.sqrt(jnp.maximum(deg_row, 1e-12))
    dc = 1.0 / jnp.sqrt(jnp.maximum(deg_col, 1e-12))
    return dc * at * dr


def _mix3(mt, h3):
    return lax.dot_general(mt, h3, (((1,), (0,)), ((), ())),
                           preferred_element_type=jnp.float32)


def _fused(x_ref, a_ref, at_ref, w1_ref, b1_ref, w2_ref, b2_ref, w3_ref,
           b3_ref, wp0_ref, bp0_ref, wp1_ref, bp1_ref, wp2_ref, bp2_ref,
           wp3_ref, bp3_ref, wp4_ref, bp4_ref,
           o0_ref, o1_ref, o2_ref, o3_ref, o4_ref, *, bs):
    bf16 = jnp.bfloat16
    mt = _norm_adj_t(a_ref[...], at_ref[...]).astype(bf16)
    w1 = w1_ref[...].astype(bf16)
    w2 = w2_ref[...].astype(bf16)

    xm = _mix3(mt, x_ref[...])
    xr = xm.reshape(N_NODES * bs, F_IN).astype(bf16)
    h1 = jnp.dot(xr, w1, preferred_element_type=jnp.float32)
    x1 = jax.nn.relu(h1 + b1_ref[...]).astype(bf16)
    h2 = jnp.dot(x1, w2, preferred_element_type=jnp.float32)
    h23 = h2.astype(bf16).reshape(N_NODES, bs, HID)
    x2m = _mix3(mt, h23)

    y = jnp.zeros((bs, FC), dtype=jnp.float32)
    for n in range(N_NODES):
        x2n = jnp.tanh(x2m[n] + b2_ref[...])
        y = y + jnp.dot(x2n.astype(bf16), w3_ref[n].astype(bf16),
                        preferred_element_type=jnp.float32)
    y = jax.nn.relu(y + b3_ref[...]).astype(bf16)
    for o_ref, w_ref, b_ref in ((o0_ref, wp0_ref, bp0_ref),
                                (o1_ref, wp1_ref, bp1_ref),
                                (o2_ref, wp2_ref, bp2_ref),
                                (o3_ref, wp3_ref, bp3_ref),
                                (o4_ref, wp4_ref, bp4_ref)):
        wk = w_ref[...].astype(bf16)
        o_ref[...] = (jnp.dot(y, wk, preferred_element_type=jnp.float32)
                      + b_ref[...])


def _full(shape):
    return pl.BlockSpec(shape, lambda *_: tuple(0 for _ in shape))


def kernel(wav_input, graph_features, graph_input, gcn1_W, gcn1_b, gcn2_W,
           gcn2_b, fc1_W, fc1_b, pga_W, pga_b, pgv_W, pgv_b, sa03_W, sa03_b,
           sa10_W, sa10_b, sa30_W, sa30_b):
    del graph_features
    B = wav_input.shape[0]
    f32 = jnp.float32
    bf16 = jnp.bfloat16
    bs = 128

    xt = jnp.transpose(wav_input, (1, 0, 2)).astype(bf16)
    a = graph_input[0]
    at = a.T
    b1 = gcn1_b.reshape(1, HID)
    b2 = gcn2_b.reshape(1, HID)
    w3 = fc1_W.reshape(N_NODES, HID, FC)
    b3 = fc1_b.reshape(1, FC)

    head_spec = [_full((FC, N_NODES)), _full((1, N_NODES))] * 5
    out_sds = jax.ShapeDtypeStruct((B, N_NODES), f32)
    out_spec = pl.BlockSpec((bs, N_NODES), lambda i: (i, 0))
    outs = pl.pallas_call(
        functools.partial(_fused, bs=bs),
        grid=(B // bs,),
        in_specs=[pl.BlockSpec((N_NODES, bs, F_IN), lambda i: (0, i, 0)),
                  _full((N_NODES, N_NODES)), _full((N_NODES, N_NODES)),
                  _full((F_IN, HID)), _full((1, HID)),
                  _full((HID, HID)), _full((1, HID)),
                  _full((N_NODES, HID, FC)), _full((1, FC))] + head_spec,
        out_specs=[out_spec] * 5,
        out_shape=[out_sds] * 5,
    )(xt, a, at, gcn1_W, b1, gcn2_W, b2, w3, b3,
      pga_W, pga_b.reshape(1, N_NODES), pgv_W, pgv_b.reshape(1, N_NODES),
      sa03_W, sa03_b.reshape(1, N_NODES), sa10_W, sa10_b.reshape(1, N_NODES),
      sa30_W, sa30_b.reshape(1, N_NODES))

    return tuple(outs)


# speedup vs baseline: 45.0860x; 1.1024x over previous
import functools

import jax
import jax.numpy as jnp
from jax import lax
from jax.experimental import pallas as pl

N_NODES = 39
F_IN = 12
HID = 64
FC = 128
N_OUT = 5 * N_NODES

def _norm_adj_t(a, at):
    deg_row = jnp.sum(a, axis=0, keepdims=True)
    deg_col = jnp.sum(at, axis=1, keepdims=True)
    dr = 1.0 / jnp.sqrt(jnp.maximum(deg_row, 1e-12))
    dc = 1.0 / jnp.sqrt(jnp.maximum(deg_col, 1e-12))
    return dc * at * dr


def _mix3(mt, h3):
    return lax.dot_general(mt, h3, (((1,), (0,)), ((), ())),
                           preferred_element_type=jnp.float32)


def _fused(x_ref, aa_ref, w1_ref, b1_ref, w2_ref, b2_ref, w3_ref,
           b3_ref, wh_ref, bh_ref, o_ref, *, bs):
    bf16 = jnp.bfloat16
    mt = _norm_adj_t(aa_ref[0], aa_ref[1]).astype(bf16)
    w1 = w1_ref[...].astype(bf16)
    w2 = w2_ref[...].astype(bf16)

    xm = _mix3(mt, x_ref[...])
    xr = xm.reshape(N_NODES * bs, F_IN).astype(bf16)
    h1 = jnp.dot(xr, w1, preferred_element_type=jnp.float32)
    x1 = jax.nn.relu(h1 + b1_ref[...]).astype(bf16)
    h2 = jnp.dot(x1, w2, preferred_element_type=jnp.float32)
    h23 = h2.astype(bf16).reshape(N_NODES, bs, HID)
    x2m = _mix3(mt, h23)

    y = jnp.zeros((bs, FC), dtype=jnp.float32)
    for n in range(N_NODES):
        x2n = jnp.tanh(x2m[n] + b2_ref[...])
        y = y + jnp.dot(x2n.astype(bf16), w3_ref[n].astype(bf16),
                        preferred_element_type=jnp.float32)
    y = jax.nn.relu(y + b3_ref[...]).astype(bf16)
    o_ref[...] = (jnp.dot(y, wh_ref[...].astype(bf16),
                          preferred_element_type=jnp.float32) + bh_ref[...])


def _full(shape):
    return pl.BlockSpec(shape, lambda *_: tuple(0 for _ in shape))


def kernel(wav_input, graph_features, graph_input, gcn1_W, gcn1_b, gcn2_W,
           gcn2_b, fc1_W, fc1_b, pga_W, pga_b, pgv_W, pgv_b, sa03_W, sa03_b,
           sa10_W, sa10_b, sa30_W, sa30_b):
    del graph_features
    B = wav_input.shape[0]
    f32 = jnp.float32
    bf16 = jnp.bfloat16
    bs = 256

    xt = jnp.transpose(wav_input, (1, 0, 2)).astype(bf16)
    a = graph_input[0]
    aa = jnp.stack([a, a.T])
    b1 = gcn1_b.reshape(1, HID)
    b2 = gcn2_b.reshape(1, HID)
    w3 = fc1_W.reshape(N_NODES, HID, FC)
    b3 = fc1_b.reshape(1, FC)
    wh = jnp.concatenate([pga_W, pgv_W, sa03_W, sa10_W, sa30_W], axis=1)
    bh = jnp.concatenate([pga_b, pgv_b, sa03_b, sa10_b, sa30_b]).reshape(1, N_OUT)

    out = pl.pallas_call(
        functools.partial(_fused, bs=bs),
        grid=(B // bs,),
        in_specs=[pl.BlockSpec((N_NODES, bs, F_IN), lambda i: (0, i, 0)),
                  _full((2, N_NODES, N_NODES)),
                  _full((F_IN, HID)), _full((1, HID)),
                  _full((HID, HID)), _full((1, HID)),
                  _full((N_NODES, HID, FC)), _full((1, FC)),
                  _full((FC, N_OUT)), _full((1, N_OUT))],
        out_specs=pl.BlockSpec((bs, N_OUT), lambda i: (i, 0)),
        out_shape=jax.ShapeDtypeStruct((B, N_OUT), f32),
    )(xt, aa, gcn1_W, b1, gcn2_W, b2, w3, b3, wh, bh)

    return (out[:, 0:39], out[:, 39:78], out[:, 78:117],
            out[:, 117:156], out[:, 156:195])
